# baseline (speedup 1.0000x reference)
import functools

import numpy as np
import jax
import jax.numpy as jnp
from jax import lax
from jax.experimental import pallas as pl
from jax.experimental.pallas import tpu as pltpu
from jax.experimental.pallas import tpu_sc as plsc

_B, _H, _W, _D = 8, 128, 128, 256
_HW = _H * _W
_NS = 32
_NST = _NS * _B
_K = 128
_NW = 32
_ROWS_PER_W = 2 * _NST * _K // _NW
_CHUNK = 64


def _threefry2x32(k1, k2, x1, x2):
    x = [x1.astype(np.uint32).copy(), x2.astype(np.uint32).copy()]
    rot = [np.array([13, 15, 26, 6], np.uint32),
           np.array([17, 29, 16, 24], np.uint32)]
    ks = [np.uint32(k1), np.uint32(k2),
          np.uint32(k1) ^ np.uint32(k2) ^ np.uint32(0x1BD11BDA)]
    with np.errstate(over="ignore"):
        x[0] = x[0] + ks[0]
        x[1] = x[1] + ks[1]
        order = [(0, ks[1], ks[2], 1), (1, ks[2], ks[0], 2),
                 (0, ks[0], ks[1], 3), (1, ks[1], ks[2], 4),
                 (0, ks[2], ks[0], 5)]
        for ridx, a0, a1, i in order:
            for r in rot[ridx]:
                x[0] = (x[0] + x[1]).astype(np.uint32)
                x[1] = ((x[1] << r) | (x[1] >> np.uint32(32 - r))).astype(np.uint32)
                x[1] = x[0] ^ x[1]
            x[0] = (x[0] + a0).astype(np.uint32)
            x[1] = (x[1] + a1 + np.uint32(i)).astype(np.uint32)
    return x[0], x[1]


def _np_randint(seed, shape, minval, maxval):
    k1, k2 = np.uint32(seed >> 32), np.uint32(seed & 0xFFFFFFFF)
    hi = np.zeros(2, np.uint32)
    lo = np.arange(2, dtype=np.uint32)
    b1, b2 = _threefry2x32(k1, k2, hi, lo)
    keys = np.stack([b1, b2], 1)

    def bits32(ka, kb):
        n = int(np.prod(shape))
        i64 = np.arange(n, dtype=np.uint64)
        chi = (i64 >> np.uint64(32)).astype(np.uint32)
        clo = (i64 & np.uint64(0xFFFFFFFF)).astype(np.uint32)
        o1, o2 = _threefry2x32(ka, kb, chi, clo)
        return (o1 ^ o2).reshape(shape)

    hb = bits32(keys[0, 0], keys[0, 1])
    lb = bits32(keys[1, 0], keys[1, 1])
    span = np.uint32(maxval - minval)
    mult = np.uint32((pow(2, 16, int(span)) ** 2) % int(span))
    with np.errstate(over="ignore"):
        off = ((hb % span) * mult + (lb % span)) % span
    return (minval + off.astype(np.int32)).astype(np.int32)


@functools.lru_cache(maxsize=1)
def _constants():
    rnd = _np_randint(1, (_B, _HW), 0, 9)
    sorted_idx = np.argsort(-rnd, axis=1, kind="stable")
    flat_pos_idx = sorted_idx[:, :_NS].reshape(-1).astype(np.int64)
    batch_idx = (np.arange(_NST) // _NS).astype(np.int32)
    py = flat_pos_idx // _W
    px = flat_pos_idx % _W
    ys = np.arange(_H)[:, None]
    xs = np.arange(_W)[None, :]
    pos_ids = np.arange(_HW, dtype=np.int64)
    sample1 = np.zeros((_NST, _K), np.int32)
    sample2 = np.zeros((_NST, _K), np.int32)
    for s in range(_NST):
        d2 = (ys - py[s]) ** 2 + (xs - px[s]) ** 2
        key = d2.reshape(-1).astype(np.int64) * _HW + pos_ids
        part = np.argpartition(key, 2 * _K)[: 2 * _K]
        order = part[np.argsort(key[part], kind="stable")]
        sample1[s] = order[:_K]
        sample2[s] = order[_K: 2 * _K]
    return batch_idx, sample1, sample2


def _make_gather(n_rows):
    mesh = plsc.VectorSubcoreMesh(core_axis_name="c", subcore_axis_name="s")
    rows_per_w = n_rows // _NW
    chunk = rows_per_w // 32
    n_chunks = 32

    @functools.partial(
        pl.kernel,
        mesh=mesh,
        out_type=(jax.ShapeDtypeStruct((n_rows, _D), jnp.float32),
                  jax.ShapeDtypeStruct((n_rows, _D), jnp.float32)),
        scratch_types=[
            pltpu.VMEM((rows_per_w,), jnp.int32),
            pltpu.VMEM((3, chunk, _D), jnp.float32),
            pltpu.VMEM((3, chunk, _D), jnp.float32),
            [pltpu.SemaphoreType.DMA] * 3,
            [pltpu.SemaphoreType.DMA] * 3,
            [pltpu.SemaphoreType.DMA] * 3,
            [pltpu.SemaphoreType.DMA] * 3,
            pltpu.SemaphoreType.DMA,
        ],
    )
    def gather_k(feat_hbm, pos_hbm, idx_hbm, f_out, p_out,
                 idx_v, fbuf, pbuf, fin, pin, fout, pout, isem):
        wid = lax.axis_index("s") * 2 + lax.axis_index("c")
        base = wid * rows_per_w
        pltpu.async_copy(idx_hbm.at[pl.ds(base, rows_per_w)], idx_v,
                         isem).wait()

        def start_in(c, b):
            idx_c = idx_v.at[pl.ds(c * chunk, chunk)]
            pltpu.async_copy(feat_hbm.at[idx_c], fbuf.at[b], fin[b])
            pltpu.async_copy(pos_hbm.at[idx_c], pbuf.at[b], pin[b])

        def wait_in(b):
            pltpu.make_async_copy(
                feat_hbm.at[pl.ds(0, chunk)], fbuf.at[b], fin[b]).wait()
            pltpu.make_async_copy(
                pos_hbm.at[pl.ds(0, chunk)], pbuf.at[b], pin[b]).wait()

        def start_out(c, b):
            off = base + c * chunk
            pltpu.async_copy(fbuf.at[b], f_out.at[pl.ds(off, chunk)], fout[b])
            pltpu.async_copy(pbuf.at[b], p_out.at[pl.ds(off, chunk)], pout[b])

        def wait_out(b):
            pltpu.make_async_copy(
                feat_hbm.at[pl.ds(0, chunk)], fbuf.at[b], fout[b]).wait()
            pltpu.make_async_copy(
                pos_hbm.at[pl.ds(0, chunk)], pbuf.at[b], pout[b]).wait()

        start_in(0, 0)
        start_in(1, 1)

        def tri_body(i, carry):
            c = i * 3

            @pl.when(i > 0)
            def _():
                wait_out(2)
            start_in(c + 2, 2)
            wait_in(0)
            start_out(c, 0)
            wait_in(1)
            start_out(c + 1, 1)
            wait_out(0)
            start_in(c + 3, 0)
            wait_in(2)
            start_out(c + 2, 2)
            wait_out(1)
            start_in(c + 4, 1)
            return carry

        n_tri = (n_chunks - 2) // 3
        lax.fori_loop(0, n_tri, tri_body, 0)
        wait_out(2)
        wait_in(0)
        start_out(n_chunks - 2, 0)
        wait_in(1)
        start_out(n_chunks - 1, 1)
        wait_out(0)
        wait_out(1)

    return gather_k


_SG = 16
_R = _SG * _K


def _attn_body(f1_r, p1_r, f2_r, p2_r, wq_r, wk_r, wv_r, slots_r, sig_r):
    f1 = f1_r[0, 0]
    p1 = p1_r[0, 0]
    f2 = f2_r[0, 0]
    p2 = p2_r[0, 0]
    wq = wq_r[...]
    wk = wk_r[...]
    wv = wv_r[...]
    scale = jnp.float32(1.0 / 16.0)
    neg = jnp.float32(-1e30)

    row_i = lax.broadcasted_iota(jnp.int32, (_SG, _R), 0)
    blk_i = lax.broadcasted_iota(jnp.int32, (_SG, _R), 1) // _K
    mask = row_i == blk_i

    def layer(s_in, f, p):
        q = jnp.dot(s_in, wq, preferred_element_type=jnp.float32)
        u = lax.dot_general(
            q, wk, (((1,), (1,)), ((), ())),
            preferred_element_type=jnp.float32)
        fp = f + p
        lg = lax.dot_general(
            u, fp, (((1,), (1,)), ((), ())),
            preferred_element_type=jnp.float32) * scale
        lgm = jnp.where(mask, lg, neg)
        m = jnp.max(lgm, axis=-1, keepdims=True)
        e = jnp.exp(lgm - m)
        attn = e / jnp.sum(e, axis=-1, keepdims=True)
        w = jnp.dot(attn, f, preferred_element_type=jnp.float32)
        delta = jnp.dot(w, wv, preferred_element_type=jnp.float32)
        lgd = jnp.concatenate(
            [lg[j:j + 1, j * _K:(j + 1) * _K] for j in range(_SG)], axis=0)
        return s_in + delta, lgd

    s0 = jnp.concatenate(
        [p1[j * _K:j * _K + 1, :] for j in range(_SG)], axis=0)
    s1, lg1 = layer(s0, f1, p1)
    s2, lg2 = layer(s1, f2, p2)
    slots_r[...] = s2
    sig = jnp.concatenate([lg1, lg2], axis=-1)
    sig_r[...] = 1.0 / (1.0 + jnp.exp(-sig))


def _make_attn(nslots):
    ngrp = nslots // _SG
    spec4 = lambda l: pl.BlockSpec((1, 1, _R, _D), lambda g: (l, g, 0, 0))
    wspec = pl.BlockSpec((_D, _D), lambda g: (0, 0))
    return pl.pallas_call(
        _attn_body,
        grid=(ngrp,),
        in_specs=[spec4(0), spec4(0), spec4(1), spec4(1),
                  wspec, wspec, wspec],
        out_specs=[pl.BlockSpec((_SG, _D), lambda g: (g, 0)),
                   pl.BlockSpec((_SG, 2 * _K), lambda g: (g, 0))],
        out_shape=[jax.ShapeDtypeStruct((nslots, _D), jnp.float32),
                   jax.ShapeDtypeStruct((nslots, 2 * _K), jnp.float32)],
    )


def _make_seg():
    mesh = plsc.VectorSubcoreMesh(core_axis_name="c", subcore_axis_name="s")
    slots_per_w = _NST // _NW

    @functools.partial(
        pl.kernel,
        mesh=mesh,
        out_type=jax.ShapeDtypeStruct((_NST, _H, _W), jnp.float32),
        scratch_types=[
            pltpu.VMEM((_H, _W), jnp.float32),
            pltpu.VMEM((2 * _K,), jnp.int32),
            pltpu.VMEM((2 * _K,), jnp.float32),
        ],
        compiler_params=pltpu.CompilerParams(needs_layout_passes=False),
    )
    def seg_k(sig0_hbm, sig1_hbm, segidx_hbm, seg_out, buf, idx_v, val_v):
        wid = lax.axis_index("s") * 2 + lax.axis_index("c")
        half = jnp.full((16,), 0.5, jnp.float32)

        def fill_row(r, carry):
            def fill_col(c, c2):
                buf[r, pl.ds(c * 16, 16)] = half
                return c2
            lax.fori_loop(0, _W // 16, fill_col, 0)
            return carry

        lax.fori_loop(0, _H, fill_row, 0)

        def make_slot_body(sig_hbm, srow_off):
            def slot_body(j, carry):
                s = wid * slots_per_w + j
                pltpu.sync_copy(segidx_hbm.at[s], idx_v)
                pltpu.sync_copy(sig_hbm.at[s - srow_off], val_v)

                def sc(i, c2):
                    iv = idx_v[pl.ds(i * 16, 16)]
                    vv = val_v[pl.ds(i * 16, 16)]
                    plsc.store_scatter(buf, [iv >> 7, iv & 127], vv)
                    return c2

                lax.fori_loop(0, 2 * _K // 16, sc, 0)
                pltpu.sync_copy(buf, seg_out.at[s])

                def unsc(i, c2):
                    iv = idx_v[pl.ds(i * 16, 16)]
                    plsc.store_scatter(buf, [iv >> 7, iv & 127], half)
                    return c2

                lax.fori_loop(0, 2 * _K // 16, unsc, 0)
                return carry
            return slot_body

        @pl.when(wid < _NW // 2)
        def _():
            lax.fori_loop(0, slots_per_w, make_slot_body(sig0_hbm, 0), 0)

        @pl.when(wid >= _NW // 2)
        def _():
            lax.fori_loop(0, slots_per_w,
                          make_slot_body(sig1_hbm, _NST // 2), 0)

    return seg_k


def kernel(features, feature_masks, pos, Wq, Wk, Wv):
    del feature_masks
    batch_idx, sample1, sample2 = _constants()

    f2d = features.reshape(_HW * _B, _D)
    p2d = pos.reshape(_HW * _B, _D)

    idx_np = np.stack([sample1, sample2], axis=0) * _B + batch_idx[None, :, None]
    hs = _NST // 2
    nrows_h = 2 * hs * _K
    gather_h = _make_gather(nrows_h)
    attn_h = _make_attn(hs)
    halves = []
    for h in range(2):
        idx_h = jnp.asarray(
            idx_np[:, h * hs:(h + 1) * hs].reshape(-1).astype(np.int32))
        f_g, p_g = gather_h(f2d, p2d, idx_h)
        f4 = f_g.reshape(2, hs // _SG, _R, _D)
        p4 = p_g.reshape(2, hs // _SG, _R, _D)
        halves.append(attn_h(f4, p4, f4, p4, Wq, Wk, Wv))
    (slots0, sig0), (slots1, sig1) = halves

    segidx = jnp.asarray(np.concatenate([sample1, sample2], axis=1))
    seg = _make_seg()(sig0, sig1, segidx)

    slots_out = jnp.concatenate([slots0, slots1], axis=0)[None]
    batch_out = jnp.broadcast_to(jnp.asarray(batch_idx)[None], (1, _NST))
    return slots_out, batch_out, seg


# speedup vs baseline: 1.0137x; 1.0137x over previous
import functools

import numpy as np
import jax
import jax.numpy as jnp
from jax import lax
from jax.experimental import pallas as pl
from jax.experimental.pallas import tpu as pltpu
from jax.experimental.pallas import tpu_sc as plsc

_B, _H, _W, _D = 8, 128, 128, 256
_HW = _H * _W
_NS = 32
_NST = _NS * _B
_K = 128
_NW = 32
_ROWS_PER_W = 2 * _NST * _K // _NW
_CHUNK = 64


def _threefry2x32(k1, k2, x1, x2):
    x = [x1.astype(np.uint32).copy(), x2.astype(np.uint32).copy()]
    rot = [np.array([13, 15, 26, 6], np.uint32),
           np.array([17, 29, 16, 24], np.uint32)]
    ks = [np.uint32(k1), np.uint32(k2),
          np.uint32(k1) ^ np.uint32(k2) ^ np.uint32(0x1BD11BDA)]
    with np.errstate(over="ignore"):
        x[0] = x[0] + ks[0]
        x[1] = x[1] + ks[1]
        order = [(0, ks[1], ks[2], 1), (1, ks[2], ks[0], 2),
                 (0, ks[0], ks[1], 3), (1, ks[1], ks[2], 4),
                 (0, ks[2], ks[0], 5)]
        for ridx, a0, a1, i in order:
            for r in rot[ridx]:
                x[0] = (x[0] + x[1]).astype(np.uint32)
                x[1] = ((x[1] << r) | (x[1] >> np.uint32(32 - r))).astype(np.uint32)
                x[1] = x[0] ^ x[1]
            x[0] = (x[0] + a0).astype(np.uint32)
            x[1] = (x[1] + a1 + np.uint32(i)).astype(np.uint32)
    return x[0], x[1]


def _np_randint(seed, shape, minval, maxval):
    k1, k2 = np.uint32(seed >> 32), np.uint32(seed & 0xFFFFFFFF)
    hi = np.zeros(2, np.uint32)
    lo = np.arange(2, dtype=np.uint32)
    b1, b2 = _threefry2x32(k1, k2, hi, lo)
    keys = np.stack([b1, b2], 1)

    def bits32(ka, kb):
        n = int(np.prod(shape))
        i64 = np.arange(n, dtype=np.uint64)
        chi = (i64 >> np.uint64(32)).astype(np.uint32)
        clo = (i64 & np.uint64(0xFFFFFFFF)).astype(np.uint32)
        o1, o2 = _threefry2x32(ka, kb, chi, clo)
        return (o1 ^ o2).reshape(shape)

    hb = bits32(keys[0, 0], keys[0, 1])
    lb = bits32(keys[1, 0], keys[1, 1])
    span = np.uint32(maxval - minval)
    mult = np.uint32((pow(2, 16, int(span)) ** 2) % int(span))
    with np.errstate(over="ignore"):
        off = ((hb % span) * mult + (lb % span)) % span
    return (minval + off.astype(np.int32)).astype(np.int32)


@functools.lru_cache(maxsize=1)
def _constants():
    rnd = _np_randint(1, (_B, _HW), 0, 9)
    sorted_idx = np.argsort(-rnd, axis=1, kind="stable")
    flat_pos_idx = sorted_idx[:, :_NS].reshape(-1).astype(np.int64)
    batch_idx = (np.arange(_NST) // _NS).astype(np.int32)
    py = flat_pos_idx // _W
    px = flat_pos_idx % _W
    ys = np.arange(_H)[:, None]
    xs = np.arange(_W)[None, :]
    pos_ids = np.arange(_HW, dtype=np.int64)
    sample1 = np.zeros((_NST, _K), np.int32)
    sample2 = np.zeros((_NST, _K), np.int32)
    for s in range(_NST):
        d2 = (ys - py[s]) ** 2 + (xs - px[s]) ** 2
        key = d2.reshape(-1).astype(np.int64) * _HW + pos_ids
        part = np.argpartition(key, 2 * _K)[: 2 * _K]
        order = part[np.argsort(key[part], kind="stable")]
        sample1[s] = order[:_K]
        sample2[s] = order[_K: 2 * _K]
    return batch_idx, sample1, sample2


def _make_gather(n_rows):
    mesh = plsc.VectorSubcoreMesh(core_axis_name="c", subcore_axis_name="s")
    rows_per_w = n_rows // _NW
    chunk = rows_per_w // 32
    n_chunks = 32

    @functools.partial(
        pl.kernel,
        mesh=mesh,
        out_type=(jax.ShapeDtypeStruct((n_rows, _D), jnp.float32),
                  jax.ShapeDtypeStruct((n_rows, _D), jnp.float32)),
        scratch_types=[
            pltpu.VMEM((rows_per_w,), jnp.int32),
            pltpu.VMEM((3, chunk, _D), jnp.float32),
            pltpu.VMEM((3, chunk, _D), jnp.float32),
            [pltpu.SemaphoreType.DMA] * 3,
            [pltpu.SemaphoreType.DMA] * 3,
            [pltpu.SemaphoreType.DMA] * 3,
            [pltpu.SemaphoreType.DMA] * 3,
            pltpu.SemaphoreType.DMA,
        ],
    )
    def gather_k(feat_hbm, pos_hbm, idx_hbm, f_out, p_out,
                 idx_v, fbuf, pbuf, fin, pin, fout, pout, isem):
        wid = lax.axis_index("s") * 2 + lax.axis_index("c")
        base = wid * rows_per_w
        pltpu.async_copy(idx_hbm.at[pl.ds(base, rows_per_w)], idx_v,
                         isem).wait()

        def start_in(c, b):
            idx_c = idx_v.at[pl.ds(c * chunk, chunk)]
            pltpu.async_copy(feat_hbm.at[idx_c], fbuf.at[b], fin[b])
            pltpu.async_copy(pos_hbm.at[idx_c], pbuf.at[b], pin[b])

        def wait_in(b):
            pltpu.make_async_copy(
                feat_hbm.at[pl.ds(0, chunk)], fbuf.at[b], fin[b]).wait()
            pltpu.make_async_copy(
                pos_hbm.at[pl.ds(0, chunk)], pbuf.at[b], pin[b]).wait()

        def start_out(c, b):
            off = base + c * chunk
            pltpu.async_copy(fbuf.at[b], f_out.at[pl.ds(off, chunk)], fout[b])
            pltpu.async_copy(pbuf.at[b], p_out.at[pl.ds(off, chunk)], pout[b])

        def wait_out(b):
            pltpu.make_async_copy(
                feat_hbm.at[pl.ds(0, chunk)], fbuf.at[b], fout[b]).wait()
            pltpu.make_async_copy(
                pos_hbm.at[pl.ds(0, chunk)], pbuf.at[b], pout[b]).wait()

        start_in(0, 0)
        start_in(1, 1)

        def tri_body(i, carry):
            c = i * 3

            @pl.when(i > 0)
            def _():
                wait_out(2)
            start_in(c + 2, 2)
            wait_in(0)
            start_out(c, 0)
            wait_in(1)
            start_out(c + 1, 1)
            wait_out(0)
            start_in(c + 3, 0)
            wait_in(2)
            start_out(c + 2, 2)
            wait_out(1)
            start_in(c + 4, 1)
            return carry

        n_tri = (n_chunks - 2) // 3
        lax.fori_loop(0, n_tri, tri_body, 0)
        wait_out(2)
        wait_in(0)
        start_out(n_chunks - 2, 0)
        wait_in(1)
        start_out(n_chunks - 1, 1)
        wait_out(0)
        wait_out(1)

    return gather_k


_SG = 16
_R = _SG * _K


def _attn_body(f1_r, p1_r, f2_r, p2_r, wq_r, wk_r, wv_r, slots_r, sig_r):
    f1 = f1_r[0, 0]
    p1 = p1_r[0, 0]
    f2 = f2_r[0, 0]
    p2 = p2_r[0, 0]
    wq = wq_r[...]
    wk = wk_r[...]
    wv = wv_r[...]
    scale = jnp.float32(1.0 / 16.0)
    neg = jnp.float32(-1e30)

    row_i = lax.broadcasted_iota(jnp.int32, (_SG, _R), 0)
    blk_i = lax.broadcasted_iota(jnp.int32, (_SG, _R), 1) // _K
    mask = row_i == blk_i

    def layer(s_in, f, p):
        q = jnp.dot(s_in, wq, preferred_element_type=jnp.float32)
        u = lax.dot_general(
            q, wk, (((1,), (1,)), ((), ())),
            preferred_element_type=jnp.float32)
        fp = f + p
        lg = lax.dot_general(
            u, fp, (((1,), (1,)), ((), ())),
            preferred_element_type=jnp.float32) * scale
        lgm = jnp.where(mask, lg, neg)
        m = jnp.max(lgm, axis=-1, keepdims=True)
        e = jnp.exp(lgm - m)
        attn = e / jnp.sum(e, axis=-1, keepdims=True)
        w = jnp.dot(attn, f, preferred_element_type=jnp.float32)
        delta = jnp.dot(w, wv, preferred_element_type=jnp.float32)
        lgd = jnp.concatenate(
            [lg[j:j + 1, j * _K:(j + 1) * _K] for j in range(_SG)], axis=0)
        return s_in + delta, lgd

    s0 = jnp.concatenate(
        [p1[j * _K:j * _K + 1, :] for j in range(_SG)], axis=0)
    s1, lg1 = layer(s0, f1, p1)
    s2, lg2 = layer(s1, f2, p2)
    slots_r[...] = s2
    sig = jnp.concatenate([lg1, lg2], axis=-1)
    sig_r[...] = 1.0 / (1.0 + jnp.exp(-sig))


def _make_attn(nslots):
    ngrp = nslots // _SG
    spec4 = lambda l: pl.BlockSpec((1, 1, _R, _D), lambda g: (l, g, 0, 0))
    wspec = pl.BlockSpec((_D, _D), lambda g: (0, 0))
    return pl.pallas_call(
        _attn_body,
        grid=(ngrp,),
        in_specs=[spec4(0), spec4(0), spec4(1), spec4(1),
                  wspec, wspec, wspec],
        out_specs=[pl.BlockSpec((_SG, _D), lambda g: (g, 0)),
                   pl.BlockSpec((_SG, 2 * _K), lambda g: (g, 0))],
        out_shape=[jax.ShapeDtypeStruct((nslots, _D), jnp.float32),
                   jax.ShapeDtypeStruct((nslots, 2 * _K), jnp.float32)],
    )


def _make_seg():
    mesh = plsc.VectorSubcoreMesh(core_axis_name="c", subcore_axis_name="s")
    slots_per_w = _NST // _NW

    @functools.partial(
        pl.kernel,
        mesh=mesh,
        out_type=jax.ShapeDtypeStruct((_NST, _H, _W), jnp.float32),
        scratch_types=[
            pltpu.VMEM((_H, _W), jnp.float32),
            pltpu.VMEM((2 * _K,), jnp.int32),
            pltpu.VMEM((2 * _K,), jnp.float32),
        ],
        compiler_params=pltpu.CompilerParams(needs_layout_passes=False),
    )
    def seg_k(sig_hbm, segidx_hbm, seg_out, buf, idx_v, val_v):
        wid = lax.axis_index("s") * 2 + lax.axis_index("c")
        half = jnp.full((16,), 0.5, jnp.float32)

        def fill_row(r, carry):
            def fill_col(c, c2):
                buf[r, pl.ds(c * 16, 16)] = half
                return c2
            lax.fori_loop(0, _W // 16, fill_col, 0)
            return carry

        lax.fori_loop(0, _H, fill_row, 0)

        def slot_body(j, carry):
            s = wid * slots_per_w + j
            pltpu.sync_copy(segidx_hbm.at[s], idx_v)
            pltpu.sync_copy(sig_hbm.at[s], val_v)

            def sc(i, c2):
                iv = idx_v[pl.ds(i * 16, 16)]
                vv = val_v[pl.ds(i * 16, 16)]
                plsc.store_scatter(buf, [iv >> 7, iv & 127], vv)
                return c2

            lax.fori_loop(0, 2 * _K // 16, sc, 0)
            pltpu.sync_copy(buf, seg_out.at[s])

            def unsc(i, c2):
                iv = idx_v[pl.ds(i * 16, 16)]
                plsc.store_scatter(buf, [iv >> 7, iv & 127], half)
                return c2

            lax.fori_loop(0, 2 * _K // 16, unsc, 0)
            return carry

        lax.fori_loop(0, slots_per_w, slot_body, 0)

    return seg_k


def kernel(features, feature_masks, pos, Wq, Wk, Wv):
    del feature_masks
    batch_idx, sample1, sample2 = _constants()

    f2d = features.reshape(_HW * _B, _D)
    p2d = pos.reshape(_HW * _B, _D)

    idx_np = np.stack([sample1, sample2], axis=0) * _B + batch_idx[None, :, None]
    idx_all = jnp.asarray(idx_np.reshape(-1).astype(np.int32))

    f_g, p_g = _make_gather(2 * _NST * _K)(f2d, p2d, idx_all)
    f4 = f_g.reshape(2, _NST // _SG, _R, _D)
    p4 = p_g.reshape(2, _NST // _SG, _R, _D)

    slots, sig = _make_attn(_NST)(f4, p4, f4, p4, Wq, Wk, Wv)

    segidx = jnp.asarray(np.concatenate([sample1, sample2], axis=1))
    seg = _make_seg()(sig, segidx)

    slots_out = slots[None]
    batch_out = jnp.broadcast_to(jnp.asarray(batch_idx)[None], (1, _NST))
    return slots_out, batch_out, seg
